# Initial kernel scaffold; baseline (speedup 1.0000x reference)
import jax
import jax.numpy as jnp
from jax.experimental import pallas as pl


def kernel(pts, timestamps, scales, grids):
    raise NotImplementedError("write your pallas kernel here")


# speedup vs baseline: 30.2445x; 30.2445x over previous
import functools
import itertools

import jax
import jax.numpy as jnp
import numpy as np
from jax import lax
from jax.experimental import pallas as pl
from jax.experimental.pallas import tpu as pltpu
from jax.experimental.pallas import tpu_sc as plsc

_BOUNDS = 1.6
_DURATION = 300.0
_COO = list(itertools.combinations(range(4), 2))

_L = 16
_P = 128
_NC = 2
_NS = 16
_NW = _NC * _NS


def _build_sc_kernel(n_pad, C, metas):
    nlev = len(metas) // 6
    D = nlev * C
    k_per_w = n_pad // (_P * _NW)
    mesh = plsc.VectorSubcoreMesh(core_axis_name="c", subcore_axis_name="s")

    @functools.partial(
        pl.kernel,
        out_type=jax.ShapeDtypeStruct((n_pad, D), jnp.float32),
        mesh=mesh,
        compiler_params=pltpu.CompilerParams(use_tc_tiling_on_sc=False),
        scratch_types=[
            pltpu.VMEM((4, _P), jnp.float32),
            pltpu.VMEM((4, _P), jnp.int32),
            pltpu.VMEM((2, _P), jnp.float32),
            pltpu.VMEM((4, _P, C), jnp.float32),
            pltpu.VMEM((_P, D), jnp.float32),
            pltpu.VMEM((len(metas), _L), jnp.float32),
            pltpu.VMEM((len(metas), _L), jnp.int32),
            pltpu.SemaphoreType.DMA,
        ],
    )
    def sc_kernel(p4t_hbm, table_hbm, pf_hbm, pi_hbm, out_hbm, coords_v,
                  idx_v, w_v, taps_v, acc_v, pf_v, pi_v, sem):
        wid = lax.axis_index("s") * _NC + lax.axis_index("c")
        pltpu.sync_copy(pf_hbm, pf_v)
        pltpu.sync_copy(pi_hbm, pi_v)

        def emit_idx(x, y, ax, bx, ay, by, w_i, xm_i, ym_i, off_i, s):
            xs = jnp.clip((x + 1.0) * ax, 0.0, bx)
            ys = jnp.clip((y + 1.0) * ay, 0.0, by)
            x0i = xs.astype(jnp.int32)
            y0i = ys.astype(jnp.int32)
            w_v[0, s] = xs - x0i.astype(jnp.float32)
            w_v[1, s] = ys - y0i.astype(jnp.float32)
            dx = jnp.minimum(x0i + 1, xm_i) - x0i
            dyw = (jnp.minimum(y0i + 1, ym_i) - y0i) * w_i
            i00 = y0i * w_i + x0i + off_i
            idx_v[0, s] = i00
            idx_v[1, s] = i00 + dx
            idx_v[2, s] = i00 + dyw
            idx_v[3, s] = i00 + dyw + dx

        def emit_combine(lvl, first):
            col = lvl * C

            def ptg_body(g, _):
                s16 = pl.ds(g * _L, _L)
                wxv = w_v[0, s16]
                wyv = w_v[1, s16]
                for j in range(_L):
                    p = g * _L + j
                    wx = wxv[j]
                    wy = wyv[j]
                    for h in range(C // _L):
                        s = pl.ds(h * _L, _L)
                        so = pl.ds(col + h * _L, _L)
                        v00 = taps_v[0, p, s]
                        v01 = taps_v[1, p, s]
                        v10 = taps_v[2, p, s]
                        v11 = taps_v[3, p, s]
                        a = v00 + wx * (v01 - v00)
                        b = v10 + wx * (v11 - v10)
                        r = a + wy * (b - a)
                        if first:
                            acc_v[p, so] = r
                        else:
                            plsc.addupdate(acc_v.at[p, so], r)

            lax.fori_loop(0, _P // _L, ptg_body, None)

        def gather_taps():
            cps = [
                pltpu.async_copy(table_hbm.at[idx_v.at[t]], taps_v.at[t], sem)
                for t in range(4)
            ]
            for cp in cps:
                cp.wait()

        def chunk_body(k, _):
            cidx = wid + k * _NW
            base = pl.multiple_of(cidx * _P, _P)
            pltpu.sync_copy(p4t_hbm.at[:, pl.ds(base, _P)], coords_v)

            for lvl in range(nlev):
                H, W, cc0, cc1, off = metas[lvl * 6]

                def grp0_body(g, _, H=H, W=W, cc0=cc0, cc1=cc1, off=off):
                    s = pl.ds(g * _L, _L)
                    emit_idx(coords_v[cc0, s], coords_v[cc1, s],
                             0.5 * (W - 1), float(W - 1),
                             0.5 * (H - 1), float(H - 1),
                             W, W - 1, H - 1, off, s)

                lax.fori_loop(0, _P // _L, grp0_body, None)
                gather_taps()
                emit_combine(lvl, first=True)

                def plane_body(pi, _, lvl=lvl):
                    row = lvl * 6 + pi
                    pf = pf_v[row, pl.ds(0, _L)]
                    pint = pi_v[row, pl.ds(0, _L)]

                    def grp_body(g, _):
                        s = pl.ds(g * _L, _L)
                        emit_idx(coords_v[pint[4], s], coords_v[pint[5], s],
                                 pf[0], pf[1], pf[2], pf[3],
                                 pint[0], pint[2], pint[3], pint[1], s)

                    lax.fori_loop(0, _P // _L, grp_body, None)
                    gather_taps()
                    emit_combine(lvl, first=False)

                lax.fori_loop(1, 6, plane_body, None)

            pltpu.sync_copy(acc_v, out_hbm.at[pl.ds(base, _P), :])

        lax.fori_loop(0, k_per_w, chunk_body, None)

    return sc_kernel


def kernel(pts, timestamps, scales, grids):
    del scales
    n = pts.shape[0]
    C = grids[0][0].shape[1]
    n_pad = -(-n // (_P * _NW)) * (_P * _NW)

    pn = (pts - _BOUNDS) * (2.0 / (-_BOUNDS - _BOUNDS)) - 1.0
    tn = 2.0 * timestamps * _DURATION / (_DURATION - 1.0) - 1.0
    p4 = jnp.concatenate([pn, tn], axis=-1)
    p4 = jnp.pad(p4, ((0, n_pad - n), (0, 0)))
    p4t = p4.T

    metas = []
    tabs = []
    off = 0
    for planes in grids:
        for ci, g in enumerate(planes):
            _, c, H, W = g.shape
            cc0, cc1 = _COO[ci]
            metas.append((H, W, cc0, cc1, off))
            off += H * W
            tabs.append(jnp.transpose(g[0], (1, 2, 0)).reshape(H * W, c))
    table = jnp.concatenate(tabs, axis=0)

    pf = np.zeros((len(metas), _L), np.float32)
    pint = np.zeros((len(metas), _L), np.int32)
    for r, (H, W, cc0, cc1, off_r) in enumerate(metas):
        pf[r, :4] = [0.5 * (W - 1), W - 1, 0.5 * (H - 1), H - 1]
        pint[r, :6] = [W, off_r, W - 1, H - 1, cc0, cc1]

    sc_kernel = _build_sc_kernel(n_pad, C, tuple(metas))
    out = sc_kernel(p4t, table, jnp.asarray(pf), jnp.asarray(pint))
    return out[:n]
